# baseline (speedup 1.0000x reference)
import functools

import jax
import jax.numpy as jnp
from jax import lax
from jax.experimental import pallas as pl
from jax.experimental.pallas import tpu as pltpu
from jax.experimental.pallas import tpu_sc as plsc

NUM_ROWS = 1000000
DIM = 32
NUM_CORES = 2
NUM_SUBCORES = 16
NUM_WORKERS = NUM_CORES * NUM_SUBCORES
ROWS_PER_WORKER = (NUM_ROWS // NUM_WORKERS) & ~7
ROWS_LAST = NUM_ROWS - (NUM_WORKERS - 1) * ROWS_PER_WORKER


@functools.partial(
    pl.kernel,
    out_type=jax.ShapeDtypeStruct((NUM_ROWS, DIM), jnp.float32),
    mesh=plsc.VectorSubcoreMesh(core_axis_name="c", subcore_axis_name="s"),
)
def _copy_all_rows(emb_hbm, out_hbm):
    wid = lax.axis_index("s") * NUM_CORES + lax.axis_index("c")
    base = wid * ROWS_PER_WORKER

    @pl.when(wid < NUM_WORKERS - 1)
    def _():
        pltpu.sync_copy(
            emb_hbm.at[pl.ds(base, ROWS_PER_WORKER)],
            out_hbm.at[pl.ds(base, ROWS_PER_WORKER)],
        )

    @pl.when(wid == NUM_WORKERS - 1)
    def _():
        pltpu.sync_copy(
            emb_hbm.at[pl.ds(base, ROWS_LAST)],
            out_hbm.at[pl.ds(base, ROWS_LAST)],
        )


def kernel(inp, embedding):
    del inp
    return _copy_all_rows(embedding)


# speedup vs baseline: 17.1918x; 17.1918x over previous
import functools

import jax
import jax.numpy as jnp
from jax import lax
from jax.experimental import pallas as pl
from jax.experimental.pallas import tpu as pltpu
from jax.experimental.pallas import tpu_sc as plsc

NUM_ROWS = 1000000
DIM = 32
NUM_CORES = 2
NUM_SUBCORES = 16
NUM_WORKERS = NUM_CORES * NUM_SUBCORES
ROWS_PER_WORKER = (NUM_ROWS // NUM_WORKERS) & ~7
CHUNK = 504
PAIRS = ROWS_PER_WORKER // (2 * CHUNK)
TAIL_BASE = NUM_WORKERS * ROWS_PER_WORKER
TAIL = NUM_ROWS - TAIL_BASE


@functools.partial(
    pl.kernel,
    out_type=jax.ShapeDtypeStruct((NUM_ROWS, DIM), jnp.float32),
    mesh=plsc.VectorSubcoreMesh(core_axis_name="c", subcore_axis_name="s"),
    scratch_types=[
        pltpu.VMEM((CHUNK, DIM), jnp.float32),
        pltpu.VMEM((CHUNK, DIM), jnp.float32),
        pltpu.SemaphoreType.DMA,
        pltpu.SemaphoreType.DMA,
        pltpu.SemaphoreType.DMA,
        pltpu.SemaphoreType.DMA,
    ],
)
def _copy_all_rows(emb, out, buf0, buf1, si0, si1, so0, so1):
    wid = lax.axis_index("s") * NUM_CORES + lax.axis_index("c")
    base = wid * ROWS_PER_WORKER

    def src(i):
        return emb.at[pl.ds(base + i * CHUNK, CHUNK)]

    def dst(i):
        return out.at[pl.ds(base + i * CHUNK, CHUNK)]

    def body(g, carry):
        i0 = 2 * g
        i1 = i0 + 1

        @pl.when(g > 0)
        def _():
            pltpu.make_async_copy(buf0, dst(i0), so0).wait()
            pltpu.make_async_copy(buf1, dst(i1), so1).wait()

        in0 = pltpu.async_copy(src(i0), buf0, si0)
        in1 = pltpu.async_copy(src(i1), buf1, si1)
        in0.wait()
        pltpu.async_copy(buf0, dst(i0), so0)
        in1.wait()
        pltpu.async_copy(buf1, dst(i1), so1)
        return carry

    lax.fori_loop(0, PAIRS, body, 0)
    pltpu.make_async_copy(buf0, dst(0), so0).wait()
    pltpu.make_async_copy(buf1, dst(1), so1).wait()

    @pl.when(wid == NUM_WORKERS - 1)
    def _():
        pltpu.sync_copy(emb.at[pl.ds(TAIL_BASE, TAIL)], buf0.at[pl.ds(0, TAIL)])
        pltpu.sync_copy(buf0.at[pl.ds(0, TAIL)], out.at[pl.ds(TAIL_BASE, TAIL)])


def kernel(inp, embedding):
    del inp
    return _copy_all_rows(embedding)


# speedup vs baseline: 147.7573x; 8.5946x over previous
import functools

import jax
import jax.numpy as jnp
from jax import lax
from jax.experimental import pallas as pl
from jax.experimental.pallas import tpu as pltpu
from jax.experimental.pallas import tpu_sc as plsc

NUM_ROWS = 1000000
DIM = 32
NUM_CORES = 2
NUM_SUBCORES = 16
NUM_WORKERS = NUM_CORES * NUM_SUBCORES
ROW_GROUPS = 4
GROUP_ROWS = DIM // ROW_GROUPS
COL_RANGES = NUM_WORKERS // ROW_GROUPS
COLS_PER_RANGE = 124928
CHUNK = 7808
PAIRS = COLS_PER_RANGE // (2 * CHUNK)
TAIL_BASE = COL_RANGES * COLS_PER_RANGE
TAIL = NUM_ROWS - TAIL_BASE


@functools.partial(
    pl.kernel,
    out_type=jax.ShapeDtypeStruct((DIM, NUM_ROWS), jnp.float32),
    mesh=plsc.VectorSubcoreMesh(core_axis_name="c", subcore_axis_name="s"),
    scratch_types=[
        pltpu.VMEM((GROUP_ROWS, CHUNK), jnp.float32),
        pltpu.VMEM((GROUP_ROWS, CHUNK), jnp.float32),
        pltpu.VMEM((GROUP_ROWS, TAIL), jnp.float32),
        pltpu.SemaphoreType.DMA,
        pltpu.SemaphoreType.DMA,
        pltpu.SemaphoreType.DMA,
        pltpu.SemaphoreType.DMA,
    ],
)
def _copy_table_t(emb, out, buf0, buf1, tailbuf, si0, si1, so0, so1):
    wid = lax.axis_index("s") * NUM_CORES + lax.axis_index("c")
    row0 = (wid // COL_RANGES) * GROUP_ROWS
    col0 = (wid % COL_RANGES) * COLS_PER_RANGE

    def src(i):
        return emb.at[pl.ds(row0, GROUP_ROWS), pl.ds(col0 + i * CHUNK, CHUNK)]

    def dst(i):
        return out.at[pl.ds(row0, GROUP_ROWS), pl.ds(col0 + i * CHUNK, CHUNK)]

    def body(g, carry):
        i0 = 2 * g
        i1 = i0 + 1

        @pl.when(g > 0)
        def _():
            pltpu.make_async_copy(buf0, dst(i0), so0).wait()
            pltpu.make_async_copy(buf1, dst(i1), so1).wait()

        in0 = pltpu.async_copy(src(i0), buf0, si0)
        in1 = pltpu.async_copy(src(i1), buf1, si1)
        in0.wait()
        pltpu.async_copy(buf0, dst(i0), so0)
        in1.wait()
        pltpu.async_copy(buf1, dst(i1), so1)
        return carry

    lax.fori_loop(0, PAIRS, body, 0)
    pltpu.make_async_copy(buf0, dst(0), so0).wait()
    pltpu.make_async_copy(buf1, dst(1), so1).wait()

    @pl.when(wid % COL_RANGES == COL_RANGES - 1)
    def _():
        pltpu.sync_copy(
            emb.at[pl.ds(row0, GROUP_ROWS), pl.ds(TAIL_BASE, TAIL)],
            tailbuf,
        )
        pltpu.sync_copy(
            tailbuf,
            out.at[pl.ds(row0, GROUP_ROWS), pl.ds(TAIL_BASE, TAIL)],
        )


def kernel(inp, embedding):
    del inp
    out_t = _copy_table_t(jnp.swapaxes(embedding, 0, 1))
    return jnp.swapaxes(out_t, 0, 1)
